# Initial kernel scaffold; baseline (speedup 1.0000x reference)
import jax
import jax.numpy as jnp
from jax.experimental import pallas as pl


def kernel(user, items, reward, user_embedding, item_embedding, umlp_w, umlp_b, imlp_w, imlp_b):
    raise NotImplementedError("write your pallas kernel here")


# speedup vs baseline: 2.0134x; 2.0134x over previous
import functools

import jax
import jax.numpy as jnp
from jax import lax
from jax.experimental import pallas as pl
from jax.experimental.pallas import tpu as pltpu
from jax.experimental.pallas import tpu_sc as plsc

_B = 4096
_L = 200
_D = 64
_REGS = 1e-05

_NC = 2
_NS = 16
_NW = _NC * _NS
_TOT = _B * _L
_PER_W = _TOT // _NW
_CH = 512
_SUB = 128
_NSUB = _CH // _SUB
_NT = _PER_W // _CH
_UPW = _B // _NW

_BB = 64
_GRID = _B // _BB


def _sc_gather(item_tab, items2d, user_tab, user_idx):
    mesh = plsc.VectorSubcoreMesh(core_axis_name="c", subcore_axis_name="s")

    @functools.partial(
        pl.kernel,
        out_type=(
            jax.ShapeDtypeStruct((_TOT, _D), jnp.float32),
            jax.ShapeDtypeStruct((_B, _D), jnp.float32),
        ),
        mesh=mesh,
        scratch_types=[
            pltpu.VMEM((_NSUB, _SUB), jnp.int32),
            pltpu.VMEM((_CH, _D), jnp.float32),
            pltpu.VMEM((_UPW,), jnp.int32),
            pltpu.VMEM((_UPW, _D), jnp.float32),
            pltpu.SemaphoreType.DMA,
            pltpu.SemaphoreType.DMA,
        ],
        compiler_params=pltpu.CompilerParams(use_tc_tiling_on_sc=False),
    )
    def k(item_hbm, iidx_hbm, user_hbm, uidx_hbm, g_out, ur_out,
          iidx_v, rows_v, uidx_v, urows_v, gsem, usem):
        cid = lax.axis_index("c")
        sid = lax.axis_index("s")
        wid = sid * _NC + cid

        ubase = wid * _UPW
        pltpu.sync_copy(uidx_hbm.at[pl.ds(ubase, _UPW)], uidx_v)
        pltpu.async_copy(user_hbm.at[uidx_v], urows_v, usem).wait()
        pltpu.sync_copy(urows_v, ur_out.at[pl.ds(ubase, _UPW)])

        base = wid * _PER_W
        idx_row_base = wid * (_PER_W // _SUB)

        def chunk(t, carry):
            pltpu.sync_copy(iidx_hbm.at[pl.ds(idx_row_base + t * _NSUB, _NSUB)],
                            iidx_v)
            descs = [
                pltpu.async_copy(item_hbm.at[iidx_v.at[j]],
                                 rows_v.at[pl.ds(j * _SUB, _SUB)], gsem)
                for j in range(_NSUB)
            ]
            for dsc in descs:
                dsc.wait()
            pltpu.sync_copy(rows_v, g_out.at[pl.ds(base + t * _CH, _CH)])
            return carry

        lax.fori_loop(0, _NT, chunk, 0)

    return k(item_tab, items2d, user_tab, user_idx)


def _sum2(x):
    return jnp.sum(jnp.sum(x, axis=1, keepdims=True), axis=0, keepdims=True)


def _tc_body(ur_ref, g_ref, rew_ref, gum_ref, uwt_ref, ub_ref, iwt_ref,
             ib_ref, gan_ref, reg_ref):
    i = pl.program_id(0)
    ue = jnp.dot(ur_ref[...], uwt_ref[...],
                 preferred_element_type=jnp.float32) + ub_ref[...]
    ie = jnp.dot(g_ref[...], iwt_ref[...],
                 preferred_element_type=jnp.float32) + ib_ref[...]
    ie3 = ie.reshape(_BB, _L, _D)
    diff = ue[:, None, :] - ie3
    d2 = jnp.sum(diff * diff, axis=-1)
    d = jnp.sqrt(d2 + 1e-12)
    m = jnp.max(d, axis=-1, keepdims=True)
    e = jnp.exp(d - m)
    s = jnp.sum(e, axis=-1, keepdims=True)
    p = e / s
    scores = jnp.log(p + 1e-12) + gum_ref[...]
    mx = jnp.max(scores, axis=-1, keepdims=True)
    iota = lax.broadcasted_iota(jnp.int32, (_BB, _L), 1)
    sel = jnp.where(scores == mx, iota, jnp.int32(2 ** 30))
    sid = jnp.min(sel, axis=-1, keepdims=True)
    onehot = iota == sid
    d_sel = jnp.sum(jnp.where(onehot, d, 0.0), axis=-1, keepdims=True)
    r_sel = jnp.sum(jnp.where(onehot, rew_ref[...], 0.0), axis=-1,
                    keepdims=True)
    lp_sel = (d_sel - m) - jnp.log(s)
    gan_part = _sum2(lp_sel * r_sel)
    reg_part = _sum2(ue * ue) + _sum2(ie * ie)

    @pl.when(i == 0)
    def _init():
        gan_ref[...] = jnp.zeros((1, 1), jnp.float32)
        reg_ref[...] = jnp.zeros((1, 1), jnp.float32)

    gan_ref[...] += gan_part
    reg_ref[...] += reg_part


def _tc_losses(ur, g2, reward, gum, uwt, ub, iwt, ib):
    return pl.pallas_call(
        _tc_body,
        grid=(_GRID,),
        in_specs=[
            pl.BlockSpec((_BB, _D), lambda i: (i, 0)),
            pl.BlockSpec((_BB * _L, _D), lambda i: (i, 0)),
            pl.BlockSpec((_BB, _L), lambda i: (i, 0)),
            pl.BlockSpec((_BB, _L), lambda i: (i, 0)),
            pl.BlockSpec((_D, _D), lambda i: (0, 0)),
            pl.BlockSpec((1, _D), lambda i: (0, 0)),
            pl.BlockSpec((_D, _D), lambda i: (0, 0)),
            pl.BlockSpec((1, _D), lambda i: (0, 0)),
        ],
        out_specs=[
            pl.BlockSpec((1, 1), lambda i: (0, 0)),
            pl.BlockSpec((1, 1), lambda i: (0, 0)),
        ],
        out_shape=[
            jax.ShapeDtypeStruct((1, 1), jnp.float32),
            jax.ShapeDtypeStruct((1, 1), jnp.float32),
        ],
    )(ur, g2, reward, gum, uwt, ub, iwt, ib)


def kernel(user, items, reward, user_embedding, item_embedding,
           umlp_w, umlp_b, imlp_w, imlp_b):
    items_flat = items.reshape(_TOT).astype(jnp.int32)
    items2d = items_flat.reshape(_TOT // _SUB, _SUB)
    user_i = user.astype(jnp.int32)

    g_rows, ur = _sc_gather(item_embedding, items2d, user_embedding, user_i)

    gum = jax.random.gumbel(jax.random.key(123), (_B, _L), jnp.float32)

    gan_sum, reg_sum = _tc_losses(
        ur, g_rows, reward, gum,
        umlp_w.T, umlp_b.reshape(1, _D),
        imlp_w.T, imlp_b.reshape(1, _D),
    )
    gan_loss = -(gan_sum[0, 0] / jnp.float32(_B))
    reg_loss = jnp.float32(_REGS * 0.5) * reg_sum[0, 0]
    return (gan_loss, reg_loss)


# speedup vs baseline: 3.6667x; 1.8212x over previous
import functools

import jax
import jax.numpy as jnp
from jax import lax
from jax.experimental import pallas as pl
from jax.experimental.pallas import tpu as pltpu
from jax.experimental.pallas import tpu_sc as plsc

_B = 4096
_L = 200
_D = 64
_REGS = 1e-05

_NC = 2
_NS = 16
_NW = _NC * _NS
_TOT = _B * _L
_PER_W = _TOT // _NW
_CH = 512
_SUB = 128
_NSUB = _CH // _SUB
_NT = _PER_W // _CH
_UPW = _B // _NW

_BB = 64
_GRID = _B // _BB


def _sc_gather(item_tab, items2d, user_tab, user_idx):
    mesh = plsc.VectorSubcoreMesh(core_axis_name="c", subcore_axis_name="s")

    @functools.partial(
        pl.kernel,
        out_type=(
            jax.ShapeDtypeStruct((_TOT, _D), jnp.float32),
            jax.ShapeDtypeStruct((_B, _D), jnp.float32),
        ),
        mesh=mesh,
        scratch_types=[
            pltpu.VMEM((_NSUB, _SUB), jnp.int32),
            pltpu.VMEM((_CH, _D), jnp.float32),
            pltpu.VMEM((_UPW,), jnp.int32),
            pltpu.VMEM((_UPW, _D), jnp.float32),
            pltpu.SemaphoreType.DMA,
            pltpu.SemaphoreType.DMA,
        ],
        compiler_params=pltpu.CompilerParams(use_tc_tiling_on_sc=False),
    )
    def k(item_hbm, iidx_hbm, user_hbm, uidx_hbm, g_out, ur_out,
          iidx_v, rows_v, uidx_v, urows_v, gsem, usem):
        cid = lax.axis_index("c")
        sid = lax.axis_index("s")
        wid = sid * _NC + cid

        ubase = wid * _UPW
        pltpu.sync_copy(uidx_hbm.at[pl.ds(ubase, _UPW)], uidx_v)
        pltpu.async_copy(user_hbm.at[uidx_v], urows_v, usem).wait()
        pltpu.sync_copy(urows_v, ur_out.at[pl.ds(ubase, _UPW)])

        base = wid * _PER_W
        idx_row_base = wid * (_PER_W // _SUB)

        def chunk(t, carry):
            pltpu.sync_copy(iidx_hbm.at[pl.ds(idx_row_base + t * _NSUB, _NSUB)],
                            iidx_v)
            descs = [
                pltpu.async_copy(item_hbm.at[iidx_v.at[j]],
                                 rows_v.at[pl.ds(j * _SUB, _SUB)], gsem)
                for j in range(_NSUB)
            ]
            for dsc in descs:
                dsc.wait()
            pltpu.sync_copy(rows_v, g_out.at[pl.ds(base + t * _CH, _CH)])
            return carry

        lax.fori_loop(0, _NT, chunk, 0)

    return k(item_tab, items2d, user_tab, user_idx)


def _sum2(x):
    return jnp.sum(jnp.sum(x, axis=1, keepdims=True), axis=0, keepdims=True)


def _tc_body(ur_ref, g_ref, rew_ref, gum_ref, bidx_ref, uwt_ref, ub_ref,
             iw_ref, ib_row_ref, ib_col_ref, gan_ref, reg_ref):
    i = pl.program_id(0)
    blk = _BB * _L
    ue = jnp.dot(ur_ref[...], uwt_ref[...],
                 preferred_element_type=jnp.float32) + ub_ref[...]
    ie_raw_t = lax.dot_general(iw_ref[...], g_ref[...],
                               (((1,), (1,)), ((), ())),
                               preferred_element_type=jnp.float32)
    uet = (ue - ib_row_ref[...]).T
    iota_b = lax.broadcasted_iota(jnp.int32, (_BB, blk), 0)
    rep = (bidx_ref[...] == iota_b).astype(jnp.float32)
    ue_rep = jnp.dot(uet, rep, preferred_element_type=jnp.float32)
    diff = ue_rep - ie_raw_t
    d2r = jnp.sum(diff * diff, axis=0, keepdims=True)
    d2g = jnp.concatenate(
        [lax.slice(d2r, (0, j * _L), (1, (j + 1) * _L))
         for j in range(_BB)], axis=0)
    d = jnp.sqrt(d2g + 1e-12)
    ie_t = ie_raw_t + ib_col_ref[...]
    e = jnp.exp(d)
    s = jnp.sum(e, axis=-1, keepdims=True)
    scores = d + gum_ref[...]
    mx = jnp.max(scores, axis=-1, keepdims=True)
    iota = lax.broadcasted_iota(jnp.int32, (_BB, _L), 1)
    sel = jnp.where(scores == mx, iota, jnp.int32(2 ** 30))
    sid = jnp.min(sel, axis=-1, keepdims=True)
    onehot = iota == sid
    d_sel = jnp.sum(jnp.where(onehot, d, 0.0), axis=-1, keepdims=True)
    r_sel = jnp.sum(jnp.where(onehot, rew_ref[...], 0.0), axis=-1,
                    keepdims=True)
    lp_sel = d_sel - jnp.log(s)
    gan_part = _sum2(lp_sel * r_sel)
    reg_part = _sum2(ue * ue) + jnp.sum(ie_t * ie_t).reshape(1, 1)

    @pl.when(i == 0)
    def _init():
        gan_ref[...] = jnp.zeros((1, 1), jnp.float32)
        reg_ref[...] = jnp.zeros((1, 1), jnp.float32)

    gan_ref[...] += gan_part
    reg_ref[...] += reg_part


def _tc_losses(ur, g2, reward, gum, bidx, uwt, ub, iw, ib_row, ib_col):
    return pl.pallas_call(
        _tc_body,
        grid=(_GRID,),
        in_specs=[
            pl.BlockSpec((_BB, _D), lambda i: (i, 0)),
            pl.BlockSpec((_BB * _L, _D), lambda i: (i, 0)),
            pl.BlockSpec((_BB, _L), lambda i: (i, 0)),
            pl.BlockSpec((_BB, _L), lambda i: (i, 0)),
            pl.BlockSpec((1, _BB * _L), lambda i: (0, 0)),
            pl.BlockSpec((_D, _D), lambda i: (0, 0)),
            pl.BlockSpec((1, _D), lambda i: (0, 0)),
            pl.BlockSpec((_D, _D), lambda i: (0, 0)),
            pl.BlockSpec((1, _D), lambda i: (0, 0)),
            pl.BlockSpec((_D, 1), lambda i: (0, 0)),
        ],
        out_specs=[
            pl.BlockSpec((1, 1), lambda i: (0, 0)),
            pl.BlockSpec((1, 1), lambda i: (0, 0)),
        ],
        out_shape=[
            jax.ShapeDtypeStruct((1, 1), jnp.float32),
            jax.ShapeDtypeStruct((1, 1), jnp.float32),
        ],
    )(ur, g2, reward, gum, bidx, uwt, ub, iw, ib_row, ib_col)


def kernel(user, items, reward, user_embedding, item_embedding,
           umlp_w, umlp_b, imlp_w, imlp_b):
    items_flat = items.reshape(_TOT).astype(jnp.int32)
    items2d = items_flat.reshape(_TOT // _SUB, _SUB)
    user_i = user.astype(jnp.int32)

    g_rows, ur = _sc_gather(item_embedding, items2d, user_embedding, user_i)

    gum = jax.random.gumbel(jax.random.key(123), (_B, _L), jnp.float32)
    bidx = (jnp.arange(_BB * _L, dtype=jnp.int32) // _L).reshape(1, _BB * _L)

    gan_sum, reg_sum = _tc_losses(
        ur, g_rows, reward, gum, bidx,
        umlp_w.T, umlp_b.reshape(1, _D),
        imlp_w, imlp_b.reshape(1, _D), imlp_b.reshape(_D, 1),
    )
    gan_loss = -(gan_sum[0, 0] / jnp.float32(_B))
    reg_loss = jnp.float32(_REGS * 0.5) * reg_sum[0, 0]
    return (gan_loss, reg_loss)


# speedup vs baseline: 6.1990x; 1.6906x over previous
import functools

import jax
import jax.numpy as jnp
import numpy as np
from jax import lax
from jax.experimental import pallas as pl
from jax.experimental.pallas import tpu as pltpu
from jax.experimental.pallas import tpu_sc as plsc

_B = 4096
_L = 200
_D = 64
_REGS = 1e-05

_NC = 2
_NS = 16
_NW = _NC * _NS
_TOT = _B * _L
_PER_W = _TOT // _NW
_CH = 640
_SUB = 128
_NSUB = _CH // _SUB
_UPW = _B // _NW
_HALF = 6400
_NPH = _PER_W // _HALF
_NTPH = _HALF // _CH

_BB = 64
_GRID = _B // _BB

def _threefry2x32_np(k1, k2, x0, x1):
    rot1 = (13, 15, 26, 6)
    rot2 = (17, 29, 16, 24)

    def rotl(x, r):
        return ((x << np.uint32(r)) | (x >> np.uint32(32 - r))).astype(np.uint32)

    ks = [k1, k2, np.uint32(k1 ^ k2 ^ np.uint32(0x1BD11BDA))]
    x0 = (x0 + ks[0]).astype(np.uint32)
    x1 = (x1 + ks[1]).astype(np.uint32)
    for i in range(5):
        for r in rot1 if i % 2 == 0 else rot2:
            x0 = (x0 + x1).astype(np.uint32)
            x1 = rotl(x1, r)
            x1 = (x1 ^ x0).astype(np.uint32)
        x0 = (x0 + ks[(i + 1) % 3]).astype(np.uint32)
        x1 = (x1 + ks[(i + 2) % 3] + np.uint32(i + 1)).astype(np.uint32)
    return x0, x1


def _gumbel_f32_np(seed, shape):
    n = int(np.prod(shape))
    o0, o1 = _threefry2x32_np(np.uint32(0), np.uint32(seed),
                              np.zeros(n, dtype=np.uint32),
                              np.arange(n, dtype=np.uint32))
    bits = o0 ^ o1
    float_bits = (bits >> np.uint32(32 - 23)) | np.uint32(0x3F800000)
    floats = float_bits.view(np.float32) - np.float32(1.0)
    tiny = np.float32(np.finfo(np.float32).tiny)
    u = np.maximum(tiny, floats * (np.float32(1.0) - tiny) + tiny)
    g = -np.log(-np.log(u.astype(np.float32)).astype(np.float32))
    return g.astype(np.float32).reshape(shape)


_GUM = _gumbel_f32_np(123, (_B, _L))
_BIDX = (np.arange(_BB * _L, dtype=np.int32) // _L).reshape(1, _BB * _L)


def _sc_gather(item_tab, items2d, user_tab, user_idx):
    mesh = plsc.VectorSubcoreMesh(core_axis_name="c", subcore_axis_name="s")

    @functools.partial(
        pl.kernel,
        out_type=(
            jax.ShapeDtypeStruct((_TOT // 2, 2 * _D), jnp.float32),
            jax.ShapeDtypeStruct((_B // 2, 2 * _D), jnp.float32),
        ),
        mesh=mesh,
        scratch_types=[
            pltpu.VMEM((_NSUB, _SUB), jnp.int32),
            pltpu.VMEM((_CH, _D), jnp.float32),
            pltpu.VMEM((_UPW,), jnp.int32),
            pltpu.VMEM((_UPW, _D), jnp.float32),
            pltpu.SemaphoreType.DMA,
            pltpu.SemaphoreType.DMA,
        ],
        compiler_params=pltpu.CompilerParams(use_tc_tiling_on_sc=False),
    )
    def k(item_hbm, iidx_hbm, user_hbm, uidx_hbm, g_out, ur_out,
          iidx_v, rows_v, uidx_v, urows_v, gsem, usem):
        cid = lax.axis_index("c")
        sid = lax.axis_index("s")
        wid = sid * _NC + cid

        ubase = wid * _UPW
        pltpu.sync_copy(uidx_hbm.at[pl.ds(ubase, _UPW)], uidx_v)
        pltpu.async_copy(user_hbm.at[uidx_v], urows_v, usem).wait()
        for q in range(4):
            dst_row = (wid * 2 + q // 2) * 32
            pltpu.sync_copy(
                urows_v.at[pl.ds(q * 32, 32)],
                ur_out.at[pl.ds(dst_row, 32), pl.ds((q % 2) * _D, _D)])

        for p in range(_NPH):
            lane0 = (p % 2) * _D
            row_base = wid * (_PER_W // 2) + (p // 2) * _HALF
            idx_base = wid * (_PER_W // _SUB) + p * (_HALF // _SUB)

            def chunk(t, carry, lane0=lane0, row_base=row_base,
                      idx_base=idx_base):
                pltpu.sync_copy(
                    iidx_hbm.at[pl.ds(idx_base + t * _NSUB, _NSUB)], iidx_v)
                descs = [
                    pltpu.async_copy(item_hbm.at[iidx_v.at[j]],
                                     rows_v.at[pl.ds(j * _SUB, _SUB)], gsem)
                    for j in range(_NSUB)
                ]
                for dsc in descs:
                    dsc.wait()
                pltpu.sync_copy(
                    rows_v,
                    g_out.at[pl.ds(row_base + t * _CH, _CH),
                             pl.ds(lane0, _D)])
                return carry

            lax.fori_loop(0, _NTPH, chunk, 0)

    return k(item_tab, items2d, user_tab, user_idx)


def _sum2(x):
    return jnp.sum(jnp.sum(x, axis=1, keepdims=True), axis=0, keepdims=True)


def _tc_body(ur_ref, g_ref, rew_ref, gum_ref, bidx_ref, uwt_ref, ub_ref,
             iw_ref, ib_row_ref, ib_col_ref, gan_ref, reg_ref):
    i = pl.program_id(0)
    blk = _BB * _L
    ur128 = ur_ref[...]
    ur = jnp.concatenate([ur128[:, :_D], ur128[:, _D:]], axis=0)
    ue = jnp.dot(ur, uwt_ref[...],
                 preferred_element_type=jnp.float32) + ub_ref[...]
    g128 = g_ref[...]
    iw = iw_ref[...]
    ie_raw_t = jnp.concatenate(
        [lax.dot_general(iw, g128[:, :_D], (((1,), (1,)), ((), ())),
                         preferred_element_type=jnp.float32),
         lax.dot_general(iw, g128[:, _D:], (((1,), (1,)), ((), ())),
                         preferred_element_type=jnp.float32)],
        axis=1)
    uet = (ue - ib_row_ref[...]).T
    iota_b = lax.broadcasted_iota(jnp.int32, (_BB, blk), 0)
    rep = (bidx_ref[...] == iota_b).astype(jnp.float32)
    ue_rep = jnp.dot(uet, rep, preferred_element_type=jnp.float32)
    diff = ue_rep - ie_raw_t
    d2r = jnp.sum(diff * diff, axis=0, keepdims=True)
    d2g = jnp.concatenate(
        [lax.slice(d2r, (0, j * _L), (1, (j + 1) * _L))
         for j in range(_BB)], axis=0)
    d = jnp.sqrt(d2g + 1e-12)
    ie_t = ie_raw_t + ib_col_ref[...]
    e = jnp.exp(d)
    s = jnp.sum(e, axis=-1, keepdims=True)
    scores = d + gum_ref[...]
    mx = jnp.max(scores, axis=-1, keepdims=True)
    iota = lax.broadcasted_iota(jnp.int32, (_BB, _L), 1)
    sel = jnp.where(scores == mx, iota, jnp.int32(2 ** 30))
    sid = jnp.min(sel, axis=-1, keepdims=True)
    onehot = iota == sid
    d_sel = jnp.sum(jnp.where(onehot, d, 0.0), axis=-1, keepdims=True)
    r_sel = jnp.sum(jnp.where(onehot, rew_ref[...], 0.0), axis=-1,
                    keepdims=True)
    lp_sel = d_sel - jnp.log(s)
    gan_part = _sum2(lp_sel * r_sel)
    reg_part = _sum2(ue * ue) + jnp.sum(ie_t * ie_t).reshape(1, 1)

    @pl.when(i == 0)
    def _init():
        gan_ref[...] = jnp.zeros((1, 1), jnp.float32)
        reg_ref[...] = jnp.zeros((1, 1), jnp.float32)

    gan_ref[...] += gan_part
    reg_ref[...] += reg_part


def _tc_losses(ur, g2, reward, gum, bidx, uwt, ub, iw, ib_row, ib_col):
    return pl.pallas_call(
        _tc_body,
        grid=(_GRID,),
        in_specs=[
            pl.BlockSpec((_BB // 2, 2 * _D), lambda i: (i, 0)),
            pl.BlockSpec((_BB * _L // 2, 2 * _D), lambda i: (i, 0)),
            pl.BlockSpec((_BB, _L), lambda i: (i, 0)),
            pl.BlockSpec((_BB, _L), lambda i: (i, 0)),
            pl.BlockSpec((1, _BB * _L), lambda i: (0, 0)),
            pl.BlockSpec((_D, _D), lambda i: (0, 0)),
            pl.BlockSpec((1, _D), lambda i: (0, 0)),
            pl.BlockSpec((_D, _D), lambda i: (0, 0)),
            pl.BlockSpec((1, _D), lambda i: (0, 0)),
            pl.BlockSpec((_D, 1), lambda i: (0, 0)),
        ],
        out_specs=[
            pl.BlockSpec((1, 1), lambda i: (0, 0)),
            pl.BlockSpec((1, 1), lambda i: (0, 0)),
        ],
        out_shape=[
            jax.ShapeDtypeStruct((1, 1), jnp.float32),
            jax.ShapeDtypeStruct((1, 1), jnp.float32),
        ],
    )(ur, g2, reward, gum, bidx, uwt, ub, iw, ib_row, ib_col)


def kernel(user, items, reward, user_embedding, item_embedding,
           umlp_w, umlp_b, imlp_w, imlp_b):
    items_flat = items.reshape(_TOT).astype(jnp.int32)
    items2d = items_flat.reshape(_TOT // _SUB, _SUB)
    user_i = user.astype(jnp.int32)

    g_rows, ur = _sc_gather(item_embedding, items2d, user_embedding, user_i)

    gan_sum, reg_sum = _tc_losses(
        ur, g_rows, reward, _GUM, _BIDX,
        umlp_w.T, umlp_b.reshape(1, _D),
        imlp_w, imlp_b.reshape(1, _D), imlp_b.reshape(_D, 1),
    )
    gan_loss = -(gan_sum[0, 0] / jnp.float32(_B))
    reg_loss = jnp.float32(_REGS * 0.5) * reg_sum[0, 0]
    return (gan_loss, reg_loss)


# speedup vs baseline: 6.8652x; 1.1075x over previous
import functools

import jax
import jax.numpy as jnp
import numpy as np
from jax import lax
from jax.experimental import pallas as pl
from jax.experimental.pallas import tpu as pltpu
from jax.experimental.pallas import tpu_sc as plsc

_B = 4096
_L = 200
_D = 64
_REGS = 1e-05

_NC = 2
_NS = 16
_NW = _NC * _NS
_TOT = _B * _L
_NH = 2
_BH = _B // _NH
_TOTH = _TOT // _NH
_PWH = _TOTH // _NW
_CH = 640
_SUB = 128
_NSUB = _CH // _SUB
_UPWH = _BH // _NW
_HALF = 6400
_NPH = _PWH // _HALF
_NTPH = _HALF // _CH

_BB = 64
_GRID = _B // _BB

def _threefry2x32_np(k1, k2, x0, x1):
    rot1 = (13, 15, 26, 6)
    rot2 = (17, 29, 16, 24)

    def rotl(x, r):
        return ((x << np.uint32(r)) | (x >> np.uint32(32 - r))).astype(np.uint32)

    ks = [k1, k2, np.uint32(k1 ^ k2 ^ np.uint32(0x1BD11BDA))]
    x0 = (x0 + ks[0]).astype(np.uint32)
    x1 = (x1 + ks[1]).astype(np.uint32)
    for i in range(5):
        for r in rot1 if i % 2 == 0 else rot2:
            x0 = (x0 + x1).astype(np.uint32)
            x1 = rotl(x1, r)
            x1 = (x1 ^ x0).astype(np.uint32)
        x0 = (x0 + ks[(i + 1) % 3]).astype(np.uint32)
        x1 = (x1 + ks[(i + 2) % 3] + np.uint32(i + 1)).astype(np.uint32)
    return x0, x1


def _gumbel_f32_np(seed, shape):
    n = int(np.prod(shape))
    o0, o1 = _threefry2x32_np(np.uint32(0), np.uint32(seed),
                              np.zeros(n, dtype=np.uint32),
                              np.arange(n, dtype=np.uint32))
    bits = o0 ^ o1
    float_bits = (bits >> np.uint32(32 - 23)) | np.uint32(0x3F800000)
    floats = float_bits.view(np.float32) - np.float32(1.0)
    tiny = np.float32(np.finfo(np.float32).tiny)
    u = np.maximum(tiny, floats * (np.float32(1.0) - tiny) + tiny)
    g = -np.log(-np.log(u.astype(np.float32)).astype(np.float32))
    return g.astype(np.float32).reshape(shape)


_GUM = _gumbel_f32_np(123, (_B, _L))
_BIDX = (np.arange(_BB * _L, dtype=np.int32) // _L).reshape(1, _BB * _L)


def _sc_gather(item_tab, items2d, user_tab, user_idx):
    mesh = plsc.VectorSubcoreMesh(core_axis_name="c", subcore_axis_name="s")

    @functools.partial(
        pl.kernel,
        out_type=(
            jax.ShapeDtypeStruct((_TOTH // 2, 2 * _D), jnp.float32),
            jax.ShapeDtypeStruct((_BH // 2, 2 * _D), jnp.float32),
        ),
        mesh=mesh,
        scratch_types=[
            pltpu.VMEM((_NSUB, _SUB), jnp.int32),
            pltpu.VMEM((_CH, _D), jnp.float32),
            pltpu.VMEM((_UPWH,), jnp.int32),
            pltpu.VMEM((_UPWH, _D), jnp.float32),
            pltpu.SemaphoreType.DMA,
            pltpu.SemaphoreType.DMA,
        ],
        compiler_params=pltpu.CompilerParams(use_tc_tiling_on_sc=False),
    )
    def k(item_hbm, iidx_hbm, user_hbm, uidx_hbm, g_out, ur_out,
          iidx_v, rows_v, uidx_v, urows_v, gsem, usem):
        cid = lax.axis_index("c")
        sid = lax.axis_index("s")
        wid = sid * _NC + cid

        ubase = wid * _UPWH
        pltpu.sync_copy(uidx_hbm.at[pl.ds(ubase, _UPWH)], uidx_v)
        pltpu.async_copy(user_hbm.at[uidx_v], urows_v, usem).wait()
        for q in range(2):
            pltpu.sync_copy(
                urows_v.at[pl.ds(q * 32, 32)],
                ur_out.at[pl.ds(wid * 32, 32), pl.ds(q * _D, _D)])

        for p in range(_NPH):
            lane0 = p * _D
            row_base = wid * _HALF
            idx_base = wid * (_PWH // _SUB) + p * (_HALF // _SUB)

            def chunk(t, carry, lane0=lane0, row_base=row_base,
                      idx_base=idx_base):
                pltpu.sync_copy(
                    iidx_hbm.at[pl.ds(idx_base + t * _NSUB, _NSUB)], iidx_v)
                descs = [
                    pltpu.async_copy(item_hbm.at[iidx_v.at[j]],
                                     rows_v.at[pl.ds(j * _SUB, _SUB)], gsem)
                    for j in range(_NSUB)
                ]
                for dsc in descs:
                    dsc.wait()
                pltpu.sync_copy(
                    rows_v,
                    g_out.at[pl.ds(row_base + t * _CH, _CH),
                             pl.ds(lane0, _D)])
                return carry

            lax.fori_loop(0, _NTPH, chunk, 0)

    return k(item_tab, items2d, user_tab, user_idx)


def _sum2(x):
    return jnp.sum(jnp.sum(x, axis=1, keepdims=True), axis=0, keepdims=True)


def _tc_body(ur_ref, g_ref, rew_ref, gum_ref, bidx_ref, uwt_ref, ub_ref,
             iw_ref, ib_row_ref, ib_col_ref, gan_ref, reg_ref):
    i = pl.program_id(0)
    blk = _BB * _L
    ur128 = ur_ref[...]
    ur = jnp.concatenate([ur128[:, :_D], ur128[:, _D:]], axis=0)
    ue = jnp.dot(ur, uwt_ref[...],
                 preferred_element_type=jnp.float32) + ub_ref[...]
    g128 = g_ref[...]
    iw = iw_ref[...]
    ie_raw_t = jnp.concatenate(
        [lax.dot_general(iw, g128[:, :_D], (((1,), (1,)), ((), ())),
                         preferred_element_type=jnp.float32),
         lax.dot_general(iw, g128[:, _D:], (((1,), (1,)), ((), ())),
                         preferred_element_type=jnp.float32)],
        axis=1)
    uet = (ue - ib_row_ref[...]).T
    iota_b = lax.broadcasted_iota(jnp.int32, (_BB, blk), 0)
    rep = (bidx_ref[...] == iota_b).astype(jnp.float32)
    ue_rep = jnp.dot(uet, rep, preferred_element_type=jnp.float32)
    diff = ue_rep - ie_raw_t
    d2r = jnp.sum(diff * diff, axis=0, keepdims=True)
    d2g = jnp.concatenate(
        [lax.slice(d2r, (0, j * _L), (1, (j + 1) * _L))
         for j in range(_BB)], axis=0)
    d = jnp.sqrt(d2g + 1e-12)
    ie_t = ie_raw_t + ib_col_ref[...]
    e = jnp.exp(d)
    s = jnp.sum(e, axis=-1, keepdims=True)
    scores = d + gum_ref[...]
    mx = jnp.max(scores, axis=-1, keepdims=True)
    iota = lax.broadcasted_iota(jnp.int32, (_BB, _L), 1)
    sel = jnp.where(scores == mx, iota, jnp.int32(2 ** 30))
    sid = jnp.min(sel, axis=-1, keepdims=True)
    onehot = iota == sid
    d_sel = jnp.sum(jnp.where(onehot, d, 0.0), axis=-1, keepdims=True)
    r_sel = jnp.sum(jnp.where(onehot, rew_ref[...], 0.0), axis=-1,
                    keepdims=True)
    lp_sel = d_sel - jnp.log(s)
    gan_part = _sum2(lp_sel * r_sel)
    reg_part = _sum2(ue * ue) + jnp.sum(ie_t * ie_t).reshape(1, 1)

    @pl.when(i == 0)
    def _init():
        gan_ref[...] = jnp.zeros((1, 1), jnp.float32)
        reg_ref[...] = jnp.zeros((1, 1), jnp.float32)

    gan_ref[...] += gan_part
    reg_ref[...] += reg_part


def _tc_losses(ur, g2, reward, gum, bidx, uwt, ub, iw, ib_row, ib_col):
    return pl.pallas_call(
        _tc_body,
        grid=(_BH // _BB,),
        in_specs=[
            pl.BlockSpec((_BB // 2, 2 * _D), lambda i: (i, 0)),
            pl.BlockSpec((_BB * _L // 2, 2 * _D), lambda i: (i, 0)),
            pl.BlockSpec((_BB, _L), lambda i: (i, 0)),
            pl.BlockSpec((_BB, _L), lambda i: (i, 0)),
            pl.BlockSpec((1, _BB * _L), lambda i: (0, 0)),
            pl.BlockSpec((_D, _D), lambda i: (0, 0)),
            pl.BlockSpec((1, _D), lambda i: (0, 0)),
            pl.BlockSpec((_D, _D), lambda i: (0, 0)),
            pl.BlockSpec((1, _D), lambda i: (0, 0)),
            pl.BlockSpec((_D, 1), lambda i: (0, 0)),
        ],
        out_specs=[
            pl.BlockSpec((1, 1), lambda i: (0, 0)),
            pl.BlockSpec((1, 1), lambda i: (0, 0)),
        ],
        out_shape=[
            jax.ShapeDtypeStruct((1, 1), jnp.float32),
            jax.ShapeDtypeStruct((1, 1), jnp.float32),
        ],
    )(ur, g2, reward, gum, bidx, uwt, ub, iw, ib_row, ib_col)


def kernel(user, items, reward, user_embedding, item_embedding,
           umlp_w, umlp_b, imlp_w, imlp_b):
    items2d = items.astype(jnp.int32).reshape(_TOT // _SUB, _SUB)
    user_i = user.astype(jnp.int32)
    uwt = umlp_w.T
    ubr = umlp_b.reshape(1, _D)
    ibr = imlp_b.reshape(1, _D)
    ibc = imlp_b.reshape(_D, 1)

    nrow = _TOTH // _SUB
    gan_sum = jnp.zeros((), jnp.float32)
    reg_sum = jnp.zeros((), jnp.float32)
    for h in range(_NH):
        g_h, ur_h = _sc_gather(
            item_embedding,
            lax.slice_in_dim(items2d, h * nrow, (h + 1) * nrow, axis=0),
            user_embedding,
            lax.slice_in_dim(user_i, h * _BH, (h + 1) * _BH, axis=0))
        gs, rs = _tc_losses(
            ur_h, g_h,
            lax.slice_in_dim(reward, h * _BH, (h + 1) * _BH, axis=0),
            _GUM[h * _BH:(h + 1) * _BH], _BIDX,
            uwt, ubr, imlp_w, ibr, ibc)
        gan_sum = gan_sum + gs[0, 0]
        reg_sum = reg_sum + rs[0, 0]

    gan_loss = -(gan_sum / jnp.float32(_B))
    reg_loss = jnp.float32(_REGS * 0.5) * reg_sum
    return (gan_loss, reg_loss)
